# baseline (speedup 1.0000x reference)
import functools

import jax
import jax.numpy as jnp
from jax import lax
from jax.experimental import pallas as pl
from jax.experimental.pallas import tpu as pltpu, tpu_sc as plsc

_KI = 128


@functools.lru_cache(maxsize=None)
def _build_gather(N, V, D):
    info = plsc.get_sparse_core_info()
    NC, NS = info.num_cores, info.num_subcores
    NW = NC * NS
    n_per_w = N // NW
    C = 640
    n_chunks = n_per_w // C
    assert n_per_w % C == 0 and N % NW == 0 and C % _KI == 0
    mesh = plsc.VectorSubcoreMesh(core_axis_name="c", subcore_axis_name="s")

    @functools.partial(
        pl.kernel,
        mesh=mesh,
        out_type=jax.ShapeDtypeStruct((N, 2 * D), jnp.float32),
        scratch_types=[
            pltpu.VMEM((n_per_w,), jnp.int32),
            pltpu.VMEM((C, D), jnp.float32),
            pltpu.VMEM((C, D), jnp.float32),
            pltpu.SemaphoreType.DMA,
            pltpu.SemaphoreType.DMA,
            pltpu.SemaphoreType.DMA,
            pltpu.SemaphoreType.DMA,
        ],
        compiler_params=pltpu.CompilerParams(use_tc_tiling_on_sc=False),
    )
    def gather_k(idx_hbm, emb_hbm, out_hbm, idx_all, rows0, rows1,
                 gsem0, gsem1, osem0, osem1):
        wid = lax.axis_index("s") * NC + lax.axis_index("c")
        base0 = wid * n_per_w
        pltpu.sync_copy(idx_hbm.at[pl.ds(base0, n_per_w)], idx_all)

        bufs = (rows0, rows1)
        gsems = (gsem0, gsem1)
        osems = (osem0, osem1)
        pend_g = [None, None]
        pend_o = [None, None]
        for g in range(n_chunks + 1):
            b = g % 2
            if g < n_chunks:
                if pend_o[b] is not None:
                    pend_o[b].wait()
                    pend_o[b] = None
                pend_g[b] = [
                    pltpu.async_copy(
                        emb_hbm.at[idx_all.at[pl.ds(g * C + k * _KI, _KI)]],
                        bufs[b].at[pl.ds(k * _KI, _KI)],
                        gsems[b],
                    )
                    for k in range(C // _KI)
                ]
            if g >= 1:
                pg, pb = g - 1, (g - 1) % 2
                for cp in pend_g[pb]:
                    cp.wait()
                pend_o[pb] = pltpu.async_copy(
                    bufs[pb],
                    out_hbm.at[pl.ds(base0 + pg * C, C), pl.ds(0, D)],
                    osems[pb])
        for b in range(2):
            if pend_o[b] is not None:
                pend_o[b].wait()

    return gather_k


def _transpose_to_physical(x3, B, H, D):
    BH, BL = B // 128, 128
    DH, DL = D // 8, 8

    def body(x_ref, y_ref):
        for bh in range(BH):
            t = x_ref[0, bh * BL:(bh + 1) * BL, :D].T
            y_ref[0, :, bh, :, :] = t.reshape(DH, DL, BL)

    y5 = pl.pallas_call(
        body,
        grid=(H,),
        in_specs=[pl.BlockSpec((1, B, 2 * D), lambda h: (h, 0, 0))],
        out_specs=pl.BlockSpec((1, DH, BH, DL, BL), lambda h: (h, 0, 0, 0, 0)),
        out_shape=jax.ShapeDtypeStruct((H, DH, BH, DL, BL), jnp.float32),
    )(x3)
    return y5


def kernel(obs, action, emb):
    B, H = obs.shape
    V, D = emb.shape
    N = B * H
    idx = obs.T.reshape(N).astype(jnp.int32)
    x3 = _build_gather(N, V, D)(idx, emb).reshape(H, B, 2 * D)
    y5 = _transpose_to_physical(x3, B, H, D)
    return y5.transpose(2, 4, 0, 1, 3).reshape(B, H, D)


# speedup vs baseline: 1.0156x; 1.0156x over previous
import functools

import jax
import jax.numpy as jnp
from jax import lax
from jax.experimental import pallas as pl
from jax.experimental.pallas import tpu as pltpu, tpu_sc as plsc

_KI = 128


@functools.lru_cache(maxsize=None)
def _build_gather(N, V, D):
    info = plsc.get_sparse_core_info()
    NC, NS = info.num_cores, info.num_subcores
    NW = NC * NS
    n_per_w = N // NW
    C = 640
    n_chunks = n_per_w // C
    assert n_per_w % C == 0 and N % NW == 0 and C % _KI == 0
    mesh = plsc.VectorSubcoreMesh(core_axis_name="c", subcore_axis_name="s")

    @functools.partial(
        pl.kernel,
        mesh=mesh,
        out_type=jax.ShapeDtypeStruct((N, 2 * D), jnp.float32),
        scratch_types=[
            pltpu.VMEM((n_per_w,), jnp.int32),
            pltpu.VMEM((C, D), jnp.float32),
            pltpu.VMEM((C, D), jnp.float32),
            pltpu.SemaphoreType.DMA,
            pltpu.SemaphoreType.DMA,
            pltpu.SemaphoreType.DMA,
            pltpu.SemaphoreType.DMA,
        ],
        compiler_params=pltpu.CompilerParams(use_tc_tiling_on_sc=False),
    )
    def gather_k(idx_hbm, emb_hbm, out_hbm, idx_all, rows0, rows1,
                 gsem0, gsem1, osem0, osem1):
        wid = lax.axis_index("s") * NC + lax.axis_index("c")
        base0 = wid * n_per_w
        pltpu.sync_copy(idx_hbm.at[pl.ds(base0, n_per_w)], idx_all)

        bufs = (rows0, rows1)
        gsems = (gsem0, gsem1)
        osems = (osem0, osem1)
        pend_g = [None, None]
        pend_o = [None, None]
        for g in range(n_chunks + 1):
            b = g % 2
            if g < n_chunks:
                if pend_o[b] is not None:
                    pend_o[b].wait()
                    pend_o[b] = None
                pend_g[b] = [
                    pltpu.async_copy(
                        emb_hbm.at[idx_all.at[pl.ds(g * C + k * _KI, _KI)]],
                        bufs[b].at[pl.ds(k * _KI, _KI)],
                        gsems[b],
                    )
                    for k in range(C // _KI)
                ]
            if g >= 1:
                pg, pb = g - 1, (g - 1) % 2
                for cp in pend_g[pb]:
                    cp.wait()
                pend_o[pb] = pltpu.async_copy(
                    bufs[pb],
                    out_hbm.at[pl.ds(base0 + pg * C, C), pl.ds(0, D)],
                    osems[pb])
        for b in range(2):
            if pend_o[b] is not None:
                pend_o[b].wait()

    return gather_k


def _transpose_to_physical(x3, B, H, D):
    BH, BL = B // 128, 128
    DH, DL = D // 8, 8

    def body(x_ref, y_ref):
        ident = jax.lax.broadcasted_iota(jnp.int32, (BL, BL), 0)
        ident = jnp.where(
            ident == jax.lax.broadcasted_iota(jnp.int32, (BL, BL), 1),
            jnp.float32(1), jnp.float32(0))
        for bh in range(BH):
            xb = x_ref[0, bh * BL:(bh + 1) * BL, :D]
            t = jax.lax.dot_general(
                xb, ident, (((0,), (0,)), ((), ())),
                preferred_element_type=jnp.float32)
            y_ref[0, :, bh, :, :] = t.reshape(DH, DL, BL)

    y5 = pl.pallas_call(
        body,
        grid=(H,),
        in_specs=[pl.BlockSpec((1, B, 2 * D), lambda h: (h, 0, 0))],
        out_specs=pl.BlockSpec((1, DH, BH, DL, BL), lambda h: (h, 0, 0, 0, 0)),
        out_shape=jax.ShapeDtypeStruct((H, DH, BH, DL, BL), jnp.float32),
    )(x3)
    return y5


def kernel(obs, action, emb):
    B, H = obs.shape
    V, D = emb.shape
    N = B * H
    idx = obs.T.reshape(N).astype(jnp.int32)
    x3 = _build_gather(N, V, D)(idx, emb).reshape(H, B, 2 * D)
    y5 = _transpose_to_physical(x3, B, H, D)
    return y5.transpose(2, 4, 0, 1, 3).reshape(B, H, D)


# speedup vs baseline: 1.0199x; 1.0043x over previous
import functools

import jax
import jax.numpy as jnp
from jax import lax
from jax.experimental import pallas as pl
from jax.experimental.pallas import tpu as pltpu, tpu_sc as plsc

_KI = 128


@functools.lru_cache(maxsize=None)
def _build_gather(N, V, D):
    info = plsc.get_sparse_core_info()
    NC, NS = info.num_cores, info.num_subcores
    NW = NC * NS
    n_per_w = N // NW
    C = 256
    NB = 4
    n_chunks = n_per_w // C
    assert n_per_w % C == 0 and N % NW == 0 and C % _KI == 0
    mesh = plsc.VectorSubcoreMesh(core_axis_name="c", subcore_axis_name="s")

    @functools.partial(
        pl.kernel,
        mesh=mesh,
        out_type=jax.ShapeDtypeStruct((N, 2 * D), jnp.float32),
        scratch_types=[
            pltpu.VMEM((n_per_w,), jnp.int32),
            [pltpu.VMEM((C, D), jnp.float32) for _ in range(NB)],
            [pltpu.SemaphoreType.DMA for _ in range(2 * NB)],
        ],
        compiler_params=pltpu.CompilerParams(use_tc_tiling_on_sc=False),
    )
    def gather_k(idx_hbm, emb_hbm, out_hbm, idx_all, bufs, sems):
        wid = lax.axis_index("s") * NC + lax.axis_index("c")
        base0 = wid * n_per_w
        pltpu.sync_copy(idx_hbm.at[pl.ds(base0, n_per_w)], idx_all)

        gsems, osems = sems[:NB], sems[NB:]
        pend_g = [None] * NB
        pend_o = [None] * NB
        for g in range(n_chunks + 2):
            b = g % NB
            if g < n_chunks:
                if pend_o[b] is not None:
                    pend_o[b].wait()
                    pend_o[b] = None
                pend_g[b] = [
                    pltpu.async_copy(
                        emb_hbm.at[idx_all.at[pl.ds(g * C + k * _KI, _KI)]],
                        bufs[b].at[pl.ds(k * _KI, _KI)],
                        gsems[b],
                    )
                    for k in range(C // _KI)
                ]
            if g >= 2:
                pg, pb = g - 2, (g - 2) % NB
                for cp in pend_g[pb]:
                    cp.wait()
                pend_o[pb] = pltpu.async_copy(
                    bufs[pb],
                    out_hbm.at[pl.ds(base0 + pg * C, C), pl.ds(0, D)],
                    osems[pb])
        for b in range(NB):
            if pend_o[b] is not None:
                pend_o[b].wait()

    return gather_k


def _transpose_to_physical(x3, B, H, D):
    BH, BL = B // 128, 128
    DH, DL = D // 8, 8

    def body(x_ref, y_ref):
        ident = jax.lax.broadcasted_iota(jnp.int32, (BL, BL), 0)
        ident = jnp.where(
            ident == jax.lax.broadcasted_iota(jnp.int32, (BL, BL), 1),
            jnp.float32(1), jnp.float32(0))
        for bh in range(BH):
            xb = x_ref[0, bh * BL:(bh + 1) * BL, :D]
            t = jax.lax.dot_general(
                xb, ident, (((0,), (0,)), ((), ())),
                preferred_element_type=jnp.float32)
            y_ref[0, :, bh, :, :] = t.reshape(DH, DL, BL)

    y5 = pl.pallas_call(
        body,
        grid=(H,),
        in_specs=[pl.BlockSpec((1, B, 2 * D), lambda h: (h, 0, 0))],
        out_specs=pl.BlockSpec((1, DH, BH, DL, BL), lambda h: (h, 0, 0, 0, 0)),
        out_shape=jax.ShapeDtypeStruct((H, DH, BH, DL, BL), jnp.float32),
    )(x3)
    return y5


def kernel(obs, action, emb):
    B, H = obs.shape
    V, D = emb.shape
    N = B * H
    idx = obs.T.reshape(N).astype(jnp.int32)
    x3 = _build_gather(N, V, D)(idx, emb).reshape(H, B, 2 * D)
    y5 = _transpose_to_physical(x3, B, H, D)
    return y5.transpose(2, 4, 0, 1, 3).reshape(B, H, D)
